# Initial kernel scaffold; baseline (speedup 1.0000x reference)
import jax
import jax.numpy as jnp
from jax.experimental import pallas as pl


def kernel(positions, mapping):
    raise NotImplementedError("write your pallas kernel here")


# speedup vs baseline: 21.4792x; 21.4792x over previous
import jax
import jax.numpy as jnp
from jax import lax
from jax.experimental import pallas as pl
from jax.experimental.pallas import tpu as pltpu
from jax.experimental.pallas import tpu_sc as plsc

N_ATOMS = 100000
N_PAIRS = 6400000
NPAD = 100096
K = 128
NW = 32
CHUNKS_PER_W = 1563
P_PAD = NW * CHUNKS_PER_W * K
ROWS_PER_TILE = NPAD // 16
R128 = (4 * NPAD) // 128
AE_ROW0 = (3 * NPAD) // 128


def _rsqrt(x):
    i = plsc.bitcast(x, jnp.int32)
    i = 0x5F3759DF - lax.shift_right_logical(i, 1)
    y = plsc.bitcast(i, jnp.float32)
    for _ in range(3):
        y = y * (1.5 - 0.5 * x * y * y)
    return y


def _sc_body(x_hbm, y_hbm, z_hbm, zero_hbm, ii_hbm, jj_hbm, part_hbm,
             x_s, y_s, z_s, fx_s, fy_s, fz_s, ae_s,
             idx_i, idx_j, xi_v, yi_v, zi_v, xj_v, yj_v, zj_v,
             fxi_v, fyi_v, fzi_v, fxj_v, fyj_v, fzj_v, e_v, stage_v, sem):
    c = lax.axis_index("c")
    s = lax.axis_index("s")

    sl = pl.ds(s * ROWS_PER_TILE, ROWS_PER_TILE)
    for src, dst in ((x_hbm, x_s), (y_hbm, y_s), (z_hbm, z_s),
                     (zero_hbm, fx_s), (zero_hbm, fy_s), (zero_hbm, fz_s),
                     (zero_hbm, ae_s)):
        pltpu.sync_copy(src.at[sl], stage_v)
        pltpu.sync_copy(stage_v, dst.at[sl])
    plsc.subcore_barrier()

    w = s * 2 + c
    base = w * (CHUNKS_PER_W * K)

    def chunk(g, carry):
        off = base + g * K
        pltpu.sync_copy(ii_hbm.at[pl.ds(off, K)], idx_i)
        pltpu.sync_copy(jj_hbm.at[pl.ds(off, K)], idx_j)
        pltpu.async_copy(x_s.at[idx_i], xi_v, sem).wait()
        pltpu.async_copy(y_s.at[idx_i], yi_v, sem).wait()
        pltpu.async_copy(z_s.at[idx_i], zi_v, sem).wait()
        pltpu.async_copy(x_s.at[idx_j], xj_v, sem).wait()
        pltpu.async_copy(y_s.at[idx_j], yj_v, sem).wait()
        pltpu.async_copy(z_s.at[idx_j], zj_v, sem).wait()
        for grp in range(K // 16):
            o = pl.ds(grp * 16, 16)
            dx = xj_v[o] - xi_v[o]
            dy = yj_v[o] - yi_v[o]
            dz = zj_v[o] - zi_v[o]
            sq = jnp.maximum(dx * dx + dy * dy + dz * dz, 1e-24)
            yv = _rsqrt(sq)
            dist = sq * yv
            t = jnp.maximum(1.0 - dist, 0.0)
            inv_d = t * yv
            fx = inv_d * dx
            fy = inv_d * dy
            fz = inv_d * dz
            fxi_v[o] = fx
            fyi_v[o] = fy
            fzi_v[o] = fz
            fxj_v[o] = -fx
            fyj_v[o] = -fy
            fzj_v[o] = -fz
            e_v[o] = 0.25 * t * t
        pltpu.sync_copy(fxi_v, fx_s.at[idx_i], add=True)
        pltpu.sync_copy(fyi_v, fy_s.at[idx_i], add=True)
        pltpu.sync_copy(fzi_v, fz_s.at[idx_i], add=True)
        pltpu.sync_copy(e_v, ae_s.at[idx_i], add=True)
        pltpu.sync_copy(fxj_v, fx_s.at[idx_j], add=True)
        pltpu.sync_copy(fyj_v, fy_s.at[idx_j], add=True)
        pltpu.sync_copy(fzj_v, fz_s.at[idx_j], add=True)
        pltpu.sync_copy(e_v, ae_s.at[idx_j], add=True)
        return carry

    lax.fori_loop(0, CHUNKS_PER_W, chunk, 0)
    plsc.subcore_barrier()

    o0 = c * (4 * NPAD) + s * ROWS_PER_TILE
    for comp, acc in enumerate((fx_s, fy_s, fz_s, ae_s)):
        pltpu.sync_copy(acc.at[sl], stage_v)
        pltpu.sync_copy(stage_v,
                        part_hbm.at[pl.ds(o0 + comp * NPAD, ROWS_PER_TILE)])


@jax.jit
def _sc_call(x, y, z, zeros1, ii, jj):
    mesh = plsc.VectorSubcoreMesh(core_axis_name="c", subcore_axis_name="s")
    table = pltpu.VMEM_SHARED((NPAD,), jnp.float32)
    buf = pltpu.VMEM((K,), jnp.float32)
    return pl.kernel(
        _sc_body,
        out_type=jax.ShapeDtypeStruct((2 * 4 * NPAD,), jnp.float32),
        mesh=mesh,
        scratch_types=[
            table, table, table, table, table, table, table,
            pltpu.VMEM((K,), jnp.int32),
            pltpu.VMEM((K,), jnp.int32),
            buf, buf, buf, buf, buf, buf,
            buf, buf, buf, buf, buf, buf, buf,
            pltpu.VMEM((ROWS_PER_TILE,), jnp.float32),
            pltpu.SemaphoreType.DMA,
        ],
        compiler_params=pltpu.CompilerParams(needs_layout_passes=False),
    )(x, y, z, zeros1, ii, jj)


def _combine_body(part_ref, out_ref, e_ref):
    total = part_ref[0] + part_ref[1]
    out_ref[...] = total
    rows = lax.broadcasted_iota(jnp.int32, (R128, 128), 0)
    cols = lax.broadcasted_iota(jnp.int32, (R128, 128), 1)
    is_real_ae = (rows >= AE_ROW0) & ((rows - AE_ROW0) * 128 + cols < N_ATOMS)
    e_ref[0, 0] = 0.5 * jnp.sum(jnp.where(is_real_ae, total, 0.0))


@jax.jit
def _combine(part):
    return pl.pallas_call(
        _combine_body,
        out_shape=(
            jax.ShapeDtypeStruct((R128, 128), jnp.float32),
            jax.ShapeDtypeStruct((1, 1), jnp.float32),
        ),
        out_specs=(
            pl.BlockSpec(memory_space=pltpu.VMEM),
            pl.BlockSpec(memory_space=pltpu.SMEM),
        ),
    )(part)


def kernel(positions, mapping):
    pos_pad = jnp.pad(positions, ((0, NPAD - N_ATOMS), (0, 0)))
    x = pos_pad[:, 0]
    y = pos_pad[:, 1]
    z = pos_pad[:, 2]
    zeros1 = jnp.zeros((NPAD,), jnp.float32)
    pad = jnp.full((P_PAD - N_PAIRS,), N_ATOMS, jnp.int32)
    ii = jnp.concatenate([mapping[0], pad])
    jj = jnp.concatenate([mapping[1], pad])
    part = _sc_call(x, y, z, zeros1, ii, jj)
    summed, e = _combine(part.reshape(2, R128, 128))
    flat = summed.reshape(4, NPAD)
    forces = jnp.stack([flat[0, :N_ATOMS], flat[1, :N_ATOMS],
                        flat[2, :N_ATOMS]], axis=1)
    atom_energies = flat[3, :N_ATOMS]
    return (e[0, 0], atom_energies, forces)


# speedup vs baseline: 39.6311x; 1.8451x over previous
import jax
import jax.numpy as jnp
from jax import lax
from jax.experimental import pallas as pl
from jax.experimental.pallas import tpu as pltpu
from jax.experimental.pallas import tpu_sc as plsc

N_ATOMS = 100000
N_PAIRS = 6400000
NPAD = 100096
K = 128
NW = 32
CHUNKS_PER_W = 1564
NH = CHUNKS_PER_W // 2
P_PAD = NW * CHUNKS_PER_W * K
ROWS_PER_TILE = NPAD // 16
R128 = (4 * NPAD) // 128
AE_ROW0 = (3 * NPAD) // 128


def _rsqrt(x):
    i = plsc.bitcast(x, jnp.int32)
    i = 0x5F3759DF - lax.shift_right_logical(i, 1)
    y = plsc.bitcast(i, jnp.float32)
    for _ in range(3):
        y = y * (1.5 - 0.5 * x * y * y)
    return y


def _sc_body(x_hbm, y_hbm, z_hbm, zero_hbm, ii_hbm, jj_hbm, part_hbm,
             x_s, y_s, z_s, fx_s, fy_s, fz_s, ae_s,
             idx_i0, idx_j0, xi0, yi0, zi0, xj0, yj0, zj0,
             fxi0, fyi0, fzi0, fxj0, fyj0, fzj0, ev0,
             idx_i1, idx_j1, xi1, yi1, zi1, xj1, yj1, zj1,
             fxi1, fyi1, fzi1, fxj1, fyj1, fzj1, ev1,
             stage_v, gsem0, gsem1, ssem0, ssem1):
    c = lax.axis_index("c")
    s = lax.axis_index("s")

    IDX_I = (idx_i0, idx_i1)
    IDX_J = (idx_j0, idx_j1)
    GI = ((xi0, yi0, zi0), (xi1, yi1, zi1))
    GJ = ((xj0, yj0, zj0), (xj1, yj1, zj1))
    UPD = ((fxi0, fyi0, fzi0, fxj0, fyj0, fzj0, ev0),
           (fxi1, fyi1, fzi1, fxj1, fyj1, fzj1, ev1))
    GSEM = (gsem0, gsem1)
    SSEM = (ssem0, ssem1)
    TABLES = (x_s, y_s, z_s)

    sl = pl.ds(s * ROWS_PER_TILE, ROWS_PER_TILE)
    for src, dst in ((x_hbm, x_s), (y_hbm, y_s), (z_hbm, z_s),
                     (zero_hbm, fx_s), (zero_hbm, fy_s), (zero_hbm, fz_s),
                     (zero_hbm, ae_s)):
        pltpu.sync_copy(src.at[sl], stage_v)
        pltpu.sync_copy(stage_v, dst.at[sl])
    plsc.subcore_barrier()

    w = s * 2 + c
    base = w * (CHUNKS_PER_W * K)

    def fetch(b, off):
        pltpu.sync_copy(ii_hbm.at[pl.ds(off, K)], IDX_I[b])
        pltpu.sync_copy(jj_hbm.at[pl.ds(off, K)], IDX_J[b])
        for t, dst in zip(TABLES, GI[b]):
            pltpu.async_copy(t.at[IDX_I[b]], dst, GSEM[b])
        for t, dst in zip(TABLES, GJ[b]):
            pltpu.async_copy(t.at[IDX_J[b]], dst, GSEM[b])

    def wait_gathers(b):
        for t, dst in zip(TABLES, GI[b]):
            pltpu.make_async_copy(t.at[IDX_I[b]], dst, GSEM[b]).wait()
        for t, dst in zip(TABLES, GJ[b]):
            pltpu.make_async_copy(t.at[IDX_J[b]], dst, GSEM[b]).wait()

    def _scatter_list(b):
        fxi, fyi, fzi, fxj, fyj, fzj, ev = UPD[b]
        return ((fxi, fx_s, IDX_I[b]), (fyi, fy_s, IDX_I[b]),
                (fzi, fz_s, IDX_I[b]), (ev, ae_s, IDX_I[b]),
                (fxj, fx_s, IDX_J[b]), (fyj, fy_s, IDX_J[b]),
                (fzj, fz_s, IDX_J[b]), (ev, ae_s, IDX_J[b]))

    def fire_scatters(b):
        for src, acc, idx in _scatter_list(b):
            pltpu.async_copy(src, acc.at[idx], SSEM[b], add=True)

    def wait_scatters(b):
        for src, acc, idx in _scatter_list(b):
            pltpu.make_async_copy(src, acc.at[idx], SSEM[b]).wait()

    def compute(b):
        xi_v, yi_v, zi_v = GI[b]
        xj_v, yj_v, zj_v = GJ[b]
        fxi_v, fyi_v, fzi_v, fxj_v, fyj_v, fzj_v, e_v = UPD[b]
        for grp in range(K // 16):
            o = pl.ds(grp * 16, 16)
            dx = xj_v[o] - xi_v[o]
            dy = yj_v[o] - yi_v[o]
            dz = zj_v[o] - zi_v[o]
            sq = jnp.maximum(dx * dx + dy * dy + dz * dz, 1e-24)
            yv = _rsqrt(sq)
            dist = sq * yv
            t = jnp.maximum(1.0 - dist, 0.0)
            inv_d = t * yv
            fx = inv_d * dx
            fy = inv_d * dy
            fz = inv_d * dz
            fxi_v[o] = fx
            fyi_v[o] = fy
            fzi_v[o] = fz
            fxj_v[o] = -fx
            fyj_v[o] = -fy
            fzj_v[o] = -fz
            e_v[o] = 0.25 * t * t

    fetch(0, base)

    def hbody(h, carry):
        wait_gathers(0)
        compute(0)

        @pl.when(h >= 1)
        def _():
            wait_scatters(1)
        fire_scatters(0)
        fetch(1, base + (2 * h + 1) * K)

        wait_gathers(1)
        compute(1)
        wait_scatters(0)
        fire_scatters(1)

        @pl.when(h < NH - 1)
        def _():
            fetch(0, base + (2 * h + 2) * K)
        return carry

    lax.fori_loop(0, NH, hbody, 0)
    wait_scatters(1)
    plsc.subcore_barrier()

    o0 = c * (4 * NPAD) + s * ROWS_PER_TILE
    for comp, acc in enumerate((fx_s, fy_s, fz_s, ae_s)):
        pltpu.sync_copy(acc.at[sl], stage_v)
        pltpu.sync_copy(stage_v,
                        part_hbm.at[pl.ds(o0 + comp * NPAD, ROWS_PER_TILE)])


@jax.jit
def _sc_call(x, y, z, zeros1, ii, jj):
    mesh = plsc.VectorSubcoreMesh(core_axis_name="c", subcore_axis_name="s")
    table = pltpu.VMEM_SHARED((NPAD,), jnp.float32)
    fbuf = pltpu.VMEM((K,), jnp.float32)
    ibuf = pltpu.VMEM((K,), jnp.int32)
    bufset = [ibuf, ibuf] + [fbuf] * 13
    return pl.kernel(
        _sc_body,
        out_type=jax.ShapeDtypeStruct((2 * 4 * NPAD,), jnp.float32),
        mesh=mesh,
        scratch_types=(
            [table] * 7 + bufset + bufset
            + [pltpu.VMEM((ROWS_PER_TILE,), jnp.float32)]
            + [pltpu.SemaphoreType.DMA] * 4
        ),
        compiler_params=pltpu.CompilerParams(needs_layout_passes=False),
    )(x, y, z, zeros1, ii, jj)


def _combine_body(part_ref, out_ref, e_ref):
    total = part_ref[0] + part_ref[1]
    out_ref[...] = total
    rows = lax.broadcasted_iota(jnp.int32, (R128, 128), 0)
    cols = lax.broadcasted_iota(jnp.int32, (R128, 128), 1)
    is_real_ae = (rows >= AE_ROW0) & ((rows - AE_ROW0) * 128 + cols < N_ATOMS)
    e_ref[0, 0] = 0.5 * jnp.sum(jnp.where(is_real_ae, total, 0.0))


@jax.jit
def _combine(part):
    return pl.pallas_call(
        _combine_body,
        out_shape=(
            jax.ShapeDtypeStruct((R128, 128), jnp.float32),
            jax.ShapeDtypeStruct((1, 1), jnp.float32),
        ),
        out_specs=(
            pl.BlockSpec(memory_space=pltpu.VMEM),
            pl.BlockSpec(memory_space=pltpu.SMEM),
        ),
    )(part)


def kernel(positions, mapping):
    pos_pad = jnp.pad(positions, ((0, NPAD - N_ATOMS), (0, 0)))
    x = pos_pad[:, 0]
    y = pos_pad[:, 1]
    z = pos_pad[:, 2]
    zeros1 = jnp.zeros((NPAD,), jnp.float32)
    pad = jnp.full((P_PAD - N_PAIRS,), N_ATOMS, jnp.int32)
    ii = jnp.concatenate([mapping[0], pad])
    jj = jnp.concatenate([mapping[1], pad])
    part = _sc_call(x, y, z, zeros1, ii, jj)
    summed, e = _combine(part.reshape(2, R128, 128))
    flat = summed.reshape(4, NPAD)
    forces = jnp.stack([flat[0, :N_ATOMS], flat[1, :N_ATOMS],
                        flat[2, :N_ATOMS]], axis=1)
    atom_energies = flat[3, :N_ATOMS]
    return (e[0, 0], atom_energies, forces)
